# baseline (speedup 1.0000x reference)
import functools

import jax
import jax.numpy as jnp
from jax import lax
from jax.experimental import pallas as pl
from jax.experimental.pallas import tpu as pltpu
from jax.experimental.pallas import tpu_sc as plsc

NROWS = 16384
NCOLS = 26
D = 16
V = 2600026
RB = 64
RP_CHUNKS = 1272
VP = RP_CHUNKS * 32 * RB
B_TOTAL = NROWS * NCOLS
PATTERN = 208
NW = 32
ROWS_PER_TILE = NROWS // NW
CH = 8
CHUNK = CH * NCOLS
NCHUNK = ROWS_PER_TILE // CH
B_PER_W = B_TOTAL // NW
K1_CHUNK = 1664
K1_NCHUNK = B_PER_W // K1_CHUNK
RP = 81248

_mesh = plsc.VectorSubcoreMesh(core_axis_name="c", subcore_axis_name="s")


def _make_index_kernel():
    @functools.partial(
        pl.kernel,
        mesh=_mesh,
        compiler_params=pltpu.CompilerParams(use_tc_tiling_on_sc=False),
        out_type=jax.ShapeDtypeStruct((B_TOTAL,), jnp.int32),
        scratch_types=[
            pltpu.VMEM((K1_CHUNK,), jnp.int32),
            pltpu.VMEM((K1_CHUNK,), jnp.int32),
            pltpu.VMEM((PATTERN,), jnp.int32),
        ],
    )
    def idx_kernel(x_hbm, pat_hbm, idx_out, xbuf, ibuf, pat_v):
        wid = lax.axis_index("c") * 16 + lax.axis_index("s")
        base = wid * B_PER_W
        pltpu.sync_copy(pat_hbm, pat_v)

        def chunk(c, carry):
            gb = base + c * K1_CHUNK
            pltpu.sync_copy(x_hbm.at[pl.ds(gb, K1_CHUNK)], xbuf)

            def add_body(v, cc):
                ph = 16 * lax.rem(v, 13)
                ibuf[pl.ds(16 * v, 16)] = (xbuf[pl.ds(16 * v, 16)]
                                           + pat_v[pl.ds(ph, 16)])
                return cc

            lax.fori_loop(0, K1_CHUNK // 16, add_body, 0)
            pltpu.sync_copy(ibuf, idx_out.at[pl.ds(gb, K1_CHUNK)])
            return carry

        lax.fori_loop(0, K1_NCHUNK, chunk, 0)

    return idx_kernel


def _make_gather_kernel():
    @functools.partial(
        pl.kernel,
        mesh=_mesh,
        compiler_params=pltpu.CompilerParams(use_tc_tiling_on_sc=True),
        out_type=jax.ShapeDtypeStruct((NROWS, NCOLS, D), jnp.float32),
        scratch_types=[
            pltpu.HBM((VP, D), jnp.float32),
            pltpu.VMEM((CHUNK,), jnp.int32),
            pltpu.VMEM((CHUNK,), jnp.int32),
            pltpu.VMEM((RB, D), jnp.float32),
            pltpu.VMEM((RB, D), jnp.float32),
            pltpu.VMEM((CHUNK, D), jnp.float32),
            pltpu.VMEM((CHUNK, D), jnp.float32),
            pltpu.SemaphoreType.DMA,
            pltpu.SemaphoreType.DMA,
            pltpu.SemaphoreType.DMA,
            pltpu.SemaphoreType.DMA,
            pltpu.SemaphoreType.REGULAR,
        ],
    )
    def gather_kernel(idx_hbm, table_hbm, out_hbm,
                      tab_lin, idx0, idx1, rb0, rb1, rows0, rows1,
                      isem0, isem1, gsem0, gsem1, bsem):
        cid = lax.axis_index("c")
        sid = lax.axis_index("s")
        wid = cid * 16 + sid

        def rp_fetch(k, nb):
            rr = (k * NW + wid) * RB
            return pltpu.async_copy(table_hbm.at[pl.ds(rr, RB)],
                                    (rb0, rb1)[nb], (isem0, isem1)[nb])

        rp_fetch(0, 0)
        rp_fetch(1, 1)

        def rp_body(k2, carry):
            k = 2 * k2

            def step(kk, nb):
                rr = (kk * NW + wid) * RB
                buf = (rb0, rb1)[nb]
                sem = (isem0, isem1)[nb]
                pltpu.make_async_copy(table_hbm.at[pl.ds(rr, RB)],
                                      buf, sem).wait()
                pltpu.sync_copy(buf, tab_lin.at[pl.ds(rr, RB)])
                rp_fetch(kk + 2, nb)

            step(k, 0)
            step(k + 1, 1)
            return carry

        lax.fori_loop(0, RP_CHUNKS // 2, rp_body, 0)
        for nb, tk in ((0, RP_CHUNKS), (1, RP_CHUNKS + 1)):
            rr = (tk * NW + wid) * RB
            pltpu.make_async_copy(table_hbm.at[pl.ds(rr, RB)],
                                  (rb0, rb1)[nb], (isem0, isem1)[nb]).wait()
        plsc.subcore_barrier()
        pltpu.core_barrier(bsem, core_axis_name="c")
        plsc.subcore_barrier()

        base = wid * B_PER_W
        i0 = wid * ROWS_PER_TILE
        idxs = (idx0, idx1)
        rows = (rows0, rows1)
        isems = (isem0, isem1)
        gsems = (gsem0, gsem1)

        def start(c, nb):
            pltpu.async_copy(idx_hbm.at[pl.ds(base + c * CHUNK, CHUNK)],
                             idxs[nb], isems[nb]).wait()
            return pltpu.async_copy(tab_lin.at[idxs[nb]], rows[nb], gsems[nb])

        handle = start(0, 0)
        for c in range(NCHUNK):
            nb = c % 2
            nxt = None
            if c + 1 < NCHUNK:
                nxt = start(c + 1, 1 - nb)
            handle.wait()
            pltpu.sync_copy(rows[nb].reshape(CH, NCOLS, D),
                            out_hbm.at[pl.ds(i0 + c * CH, CH)])
            handle = nxt

    return gather_kernel


_idx_k = _make_index_kernel()
_gather_k = _make_gather_kernel()


@jax.jit
def kernel(x_cat, category_offsets, table):
    x_flat = x_cat.reshape(B_TOTAL).astype(jnp.int32)
    pat = jnp.tile(category_offsets.astype(jnp.int32), PATTERN // NCOLS)
    idx = _idx_k(x_flat, pat)
    return _gather_k(idx, table)


# speedup vs baseline: 1.2570x; 1.2570x over previous
import functools

import jax
import jax.numpy as jnp
from jax import lax
from jax.experimental import pallas as pl
from jax.experimental.pallas import tpu as pltpu
from jax.experimental.pallas import tpu_sc as plsc

NCOLS = 26
NROWS = 16384
TOKEN_DIM = 16
B_TOTAL = NROWS * NCOLS
PATTERN = 208

_info = plsc.get_sparse_core_info()
NW = _info.num_cores * _info.num_subcores
B_PER_W = B_TOTAL // NW
CHUNK = 1664
NCHUNK = B_PER_W // CHUNK
VREGS_PER_CHUNK = CHUNK // 16


def _make_kernel():
    mesh = plsc.VectorSubcoreMesh(core_axis_name="c", subcore_axis_name="s")

    @functools.partial(
        pl.kernel,
        mesh=mesh,
        compiler_params=pltpu.CompilerParams(use_tc_tiling_on_sc=False),
        out_type=jax.ShapeDtypeStruct((B_TOTAL, TOKEN_DIM), jnp.float32),
        scratch_types=[
            pltpu.VMEM((CHUNK,), jnp.int32),
            pltpu.VMEM((CHUNK,), jnp.int32),
            pltpu.VMEM((CHUNK,), jnp.int32),
            pltpu.VMEM((CHUNK,), jnp.int32),
            pltpu.VMEM((CHUNK, TOKEN_DIM), jnp.float32),
            pltpu.VMEM((CHUNK, TOKEN_DIM), jnp.float32),
            pltpu.VMEM((PATTERN,), jnp.int32),
            pltpu.SemaphoreType.DMA,
            pltpu.SemaphoreType.DMA,
        ],
    )
    def emb_kernel(x_hbm, pat_hbm, table_hbm, out_hbm,
                   xbuf0, xbuf1, idxbuf0, idxbuf1, rows0, rows1,
                   pat_v, sem0, sem1):
        wid = lax.axis_index("c") * _info.num_subcores + lax.axis_index("s")
        base = wid * B_PER_W

        xbufs = (xbuf0, xbuf1)
        idxbufs = (idxbuf0, idxbuf1)
        rowbufs = (rows0, rows1)
        sems = (sem0, sem1)

        pltpu.sync_copy(pat_hbm, pat_v)

        def start_chunk(c, nb):
            gb = base + c * CHUNK
            xb, ib = xbufs[nb], idxbufs[nb]
            pltpu.sync_copy(x_hbm.at[pl.ds(gb, CHUNK)], xb)

            def add_body(v, carry):
                ph = 16 * lax.rem(v, 13)
                ib[pl.ds(16 * v, 16)] = xb[pl.ds(16 * v, 16)] + pat_v[pl.ds(ph, 16)]
                return carry

            lax.fori_loop(0, VREGS_PER_CHUNK, add_body, 0)
            return pltpu.async_copy(table_hbm.at[ib], rowbufs[nb], sems[nb])

        handle = start_chunk(0, 0)
        for c in range(NCHUNK):
            nb = c % 2
            nxt = None
            if c + 1 < NCHUNK:
                nxt = start_chunk(c + 1, 1 - nb)
            handle.wait()
            pltpu.sync_copy(rowbufs[nb],
                            out_hbm.at[pl.ds(base + c * CHUNK, CHUNK)])
            handle = nxt

    return emb_kernel


_emb = _make_kernel()


@jax.jit
def kernel(x_cat, category_offsets, table):
    x_flat = x_cat.reshape(B_TOTAL).astype(jnp.int32)
    pat = jnp.tile(category_offsets.astype(jnp.int32), PATTERN // NCOLS)
    out = _emb(x_flat, pat, table)
    return out.reshape(NROWS, NCOLS, TOKEN_DIM)


# speedup vs baseline: 1.4293x; 1.1371x over previous
import functools

import jax
import jax.numpy as jnp
from jax import lax
from jax.experimental import pallas as pl
from jax.experimental.pallas import tpu as pltpu
from jax.experimental.pallas import tpu_sc as plsc

NCOLS = 26
NROWS = 16384
TOKEN_DIM = 16
B_TOTAL = NROWS * NCOLS
PATTERN = 208

_info = plsc.get_sparse_core_info()
NW = _info.num_cores * _info.num_subcores
B_PER_W = B_TOTAL // NW
CHUNK = 1664
NCHUNK = B_PER_W // CHUNK
VREGS_PER_CHUNK = CHUNK // 16


def _make_kernel():
    mesh = plsc.VectorSubcoreMesh(core_axis_name="c", subcore_axis_name="s")

    @functools.partial(
        pl.kernel,
        mesh=mesh,
        compiler_params=pltpu.CompilerParams(use_tc_tiling_on_sc=False),
        out_type=jax.ShapeDtypeStruct((NROWS, NCOLS, TOKEN_DIM), jnp.float32),
        scratch_types=[
            pltpu.VMEM((CHUNK,), jnp.int32),
            pltpu.VMEM((CHUNK,), jnp.int32),
            pltpu.VMEM((CHUNK,), jnp.int32),
            pltpu.VMEM((CHUNK,), jnp.int32),
            pltpu.VMEM((CHUNK, TOKEN_DIM), jnp.float32),
            pltpu.VMEM((CHUNK, TOKEN_DIM), jnp.float32),
            pltpu.VMEM((PATTERN,), jnp.int32),
            pltpu.SemaphoreType.DMA,
            pltpu.SemaphoreType.DMA,
        ],
    )
    def emb_kernel(x_hbm, pat_hbm, table_hbm, out_hbm,
                   xbuf0, xbuf1, idxbuf0, idxbuf1, rows0, rows1,
                   pat_v, sem0, sem1):
        wid = lax.axis_index("c") * _info.num_subcores + lax.axis_index("s")
        base = wid * B_PER_W

        xbufs = (xbuf0, xbuf1)
        idxbufs = (idxbuf0, idxbuf1)
        rowbufs = (rows0, rows1)
        sems = (sem0, sem1)

        pltpu.sync_copy(pat_hbm, pat_v)

        def start_chunk(c, nb):
            gb = base + c * CHUNK
            xb, ib = xbufs[nb], idxbufs[nb]
            pltpu.sync_copy(x_hbm.at[pl.ds(gb, CHUNK)], xb)

            def add_body(v, carry):
                ph = 16 * lax.rem(v, 13)
                ib[pl.ds(16 * v, 16)] = xb[pl.ds(16 * v, 16)] + pat_v[pl.ds(ph, 16)]
                return carry

            lax.fori_loop(0, VREGS_PER_CHUNK, add_body, 0)
            return pltpu.async_copy(table_hbm.at[ib], rowbufs[nb], sems[nb])

        handle = start_chunk(0, 0)
        for c in range(NCHUNK):
            nb = c % 2
            nxt = None
            if c + 1 < NCHUNK:
                nxt = start_chunk(c + 1, 1 - nb)
            handle.wait()
            row0 = (base + c * CHUNK) // NCOLS

            def out_row(r, carry):
                pltpu.sync_copy(rowbufs[nb].at[pl.ds(NCOLS * r, NCOLS)],
                                out_hbm.at[row0 + r])
                return carry

            lax.fori_loop(0, CHUNK // NCOLS, out_row, 0)
            handle = nxt

    return emb_kernel


_emb = _make_kernel()


@jax.jit
def kernel(x_cat, category_offsets, table):
    x_flat = x_cat.reshape(B_TOTAL).astype(jnp.int32)
    pat = jnp.tile(category_offsets.astype(jnp.int32), PATTERN // NCOLS)
    return _emb(x_flat, pat, table)
